# baseline (speedup 1.0000x reference)
import jax
import jax.numpy as jnp
from jax.experimental import pallas as pl
from jax.experimental.pallas import tpu as pltpu

T = 16384
D = 4096
E = 64
K = 8
CHUNK = 512
NBUF = 3
NCHUNK = T // CHUNK


def _router_kernel(x_hbm, w_ref, bias_ref, out_ref, buf, sem):
    w = w_ref[...]
    bias = bias_ref[0:1, :]

    def start(c):
        slot = jax.lax.rem(c, NBUF)
        pltpu.make_async_copy(
            x_hbm.at[pl.ds(c * CHUNK, CHUNK), :], buf.at[slot], sem.at[slot]
        ).start()

    for j in range(NBUF):
        start(j)

    def body(i, carry):
        slot = jax.lax.rem(i, NBUF)
        pltpu.make_async_copy(
            x_hbm.at[pl.ds(i * CHUNK, CHUNK), :], buf.at[slot], sem.at[slot]
        ).wait()
        xblk = buf[slot]
        logits = jnp.dot(xblk, w, preferred_element_type=jnp.float32) + bias
        cur = logits
        m0 = None
        for _ in range(K):
            m = jnp.max(cur, axis=1, keepdims=True)
            if m0 is None:
                m0 = m
            cur = jnp.where(cur == m, -jnp.inf, cur)
        ex = jnp.where(cur < logits, jnp.exp(logits - m0), 0.0)
        z = jnp.sum(ex, axis=1, keepdims=True)
        out_ref[pl.ds(i * CHUNK, CHUNK), :] = ex / z

        nxt = i + NBUF

        @pl.when(nxt < NCHUNK)
        def _():
            start(nxt)

        return carry

    jax.lax.fori_loop(0, NCHUNK, body, 0)


@jax.jit
def kernel(x, w_gate, b_gate, expert_biases):
    bias = jnp.broadcast_to((b_gate + expert_biases)[None, :], (8, E))
    return pl.pallas_call(
        _router_kernel,
        in_specs=[
            pl.BlockSpec(memory_space=pltpu.MemorySpace.HBM),
            pl.BlockSpec(memory_space=pltpu.MemorySpace.VMEM),
            pl.BlockSpec(memory_space=pltpu.MemorySpace.VMEM),
        ],
        out_specs=pl.BlockSpec(memory_space=pltpu.MemorySpace.VMEM),
        out_shape=jax.ShapeDtypeStruct((T, E), x.dtype),
        scratch_shapes=[
            pltpu.VMEM((NBUF, CHUNK, D), jnp.float32),
            pltpu.SemaphoreType.DMA((NBUF,)),
        ],
    )(x, w_gate, bias)
